# Initial kernel scaffold; baseline (speedup 1.0000x reference)
import jax
import jax.numpy as jnp
from jax.experimental import pallas as pl


def kernel(x, edge_index, W1, b1, W2, b2):
    raise NotImplementedError("write your pallas kernel here")


# speedup vs baseline: 8.1378x; 8.1378x over previous
import functools

import jax
import jax.numpy as jnp
from jax import lax
from jax.experimental import pallas as pl
from jax.experimental.pallas import tpu as pltpu
from jax.experimental.pallas import tpu_sc as plsc

N = 10000
NT = 10240
E = 320000
EP = 327680
EROWS = EP // 128
ROWS_PER_TILE = EROWS // 16
DEG_ROWS_PER_TILE = EROWS // 32
NODES_PER_TILE = NT // 16

_MESH = plsc.VectorSubcoreMesh(core_axis_name="c", subcore_axis_name="s")


@functools.partial(
    pl.kernel,
    out_type=jax.ShapeDtypeStruct((2, NT), jnp.float32),
    mesh=_MESH,
    scratch_types=[
        pltpu.VMEM_SHARED((NT,), jnp.float32),
        pltpu.VMEM((DEG_ROWS_PER_TILE, 128), jnp.int32),
        pltpu.VMEM((128,), jnp.float32),
    ],
)
def _deg_kernel(dst_hbm, ones_hbm, zeros1_hbm, deg_out, deg_sh, idx_v, ones_v):
    c = lax.axis_index("c")
    s = lax.axis_index("s")
    pltpu.sync_copy(zeros1_hbm, deg_sh.at[pl.ds(s * NODES_PER_TILE, NODES_PER_TILE)])
    row0 = c * (EROWS // 2) + s * DEG_ROWS_PER_TILE
    pltpu.sync_copy(dst_hbm.at[pl.ds(row0, DEG_ROWS_PER_TILE)], idx_v)
    pltpu.sync_copy(ones_hbm, ones_v)
    plsc.subcore_barrier()

    def body(j, carry):
        pltpu.sync_copy(ones_v, deg_sh.at[idx_v.at[j]], add=True)
        return carry

    lax.fori_loop(0, DEG_ROWS_PER_TILE, body, 0)
    plsc.subcore_barrier()
    pltpu.sync_copy(
        deg_sh.at[pl.ds(s * NODES_PER_TILE, NODES_PER_TILE)],
        deg_out.at[c, pl.ds(s * NODES_PER_TILE, NODES_PER_TILE)],
    )


SLAB = 16


def _make_edge_kernel(table_rows, edge_split):
    rows_per_tile = ROWS_PER_TILE if not edge_split else EROWS // 32
    nslab = rows_per_tile // SLAB

    @functools.partial(
        pl.kernel,
        out_type=jax.ShapeDtypeStruct((2, NT, 128), jnp.float32),
        mesh=_MESH,
        scratch_types=[
            pltpu.VMEM_SHARED((NT, 128), jnp.float32),
            pltpu.VMEM((SLAB, 128), jnp.int32),
            pltpu.VMEM((SLAB, 128), jnp.int32),
            pltpu.VMEM((128, 128), jnp.float32),
            pltpu.SemaphoreType.DMA,
        ],
    )
    def edge_kernel(table_hbm, src2_hbm, dst_hbm, zeros_hbm, acc_out,
                    acc_sh, src_idx, dst_idx, rows_v, sem):
        c = lax.axis_index("c")
        s = lax.axis_index("s")
        pltpu.sync_copy(zeros_hbm, acc_sh.at[pl.ds(s * NODES_PER_TILE, NODES_PER_TILE)])
        plsc.subcore_barrier()
        if edge_split:
            r0 = c * (EROWS // 2) + s * rows_per_tile
        else:
            r0 = s * rows_per_tile

        def slab_body(t, carry):
            if edge_split:
                pltpu.sync_copy(src2_hbm.at[pl.ds(r0 + t * SLAB, SLAB)], src_idx)
            else:
                pltpu.sync_copy(src2_hbm.at[c, pl.ds(r0 + t * SLAB, SLAB)], src_idx)
            pltpu.sync_copy(dst_hbm.at[pl.ds(r0 + t * SLAB, SLAB)], dst_idx)

            def body(j, carry2):
                pltpu.async_copy(table_hbm.at[src_idx.at[j]], rows_v, sem).wait()
                pltpu.sync_copy(rows_v, acc_sh.at[dst_idx.at[j]], add=True)
                return carry2

            lax.fori_loop(0, SLAB, body, 0)
            return carry

        lax.fori_loop(0, nslab, slab_body, 0)
        plsc.subcore_barrier()
        pltpu.sync_copy(
            acc_sh.at[pl.ds(s * NODES_PER_TILE, NODES_PER_TILE)],
            acc_out.at[c, pl.ds(s * NODES_PER_TILE, NODES_PER_TILE)],
        )

    return edge_kernel


_edge_kernel_l1 = _make_edge_kernel(2 * NT, edge_split=False)
_edge_kernel_l2 = _make_edge_kernel(NT, edge_split=True)


_BLK = 512
_GRID = NT // _BLK


def _tc1_body(x_ref, w1_ref, deg_ref, hs_ref):
    d = deg_ref[:, 0:1] + deg_ref[:, 1:2] + 1.0
    dinv = lax.rsqrt(d)
    h = jnp.dot(x_ref[...], w1_ref[...], preferred_element_type=jnp.float32)
    hsc = h * dinv
    hs_ref[0] = hsc[:, :128]
    hs_ref[1] = hsc[:, 128:]


def _tc1(x_pad, W1, degT):
    return pl.pallas_call(
        _tc1_body,
        grid=(_GRID,),
        in_specs=[
            pl.BlockSpec((_BLK, 128), lambda i: (i, 0)),
            pl.BlockSpec((128, 256), lambda i: (0, 0)),
            pl.BlockSpec((_BLK, 2), lambda i: (i, 0)),
        ],
        out_specs=pl.BlockSpec((2, _BLK, 128), lambda i: (0, i, 0)),
        out_shape=jax.ShapeDtypeStruct((2, NT, 128), jnp.float32),
    )(x_pad, W1, degT)


def _tc2_body(acc_ref, hs_ref, deg_ref, b1_ref, w2_ref, h2s_ref):
    d = deg_ref[:, 0:1] + deg_ref[:, 1:2] + 1.0
    dinv = lax.rsqrt(d)
    u = jnp.concatenate([acc_ref[0] + hs_ref[0], acc_ref[1] + hs_ref[1]], axis=1)
    out1 = dinv * u + b1_ref[...]
    r = jnp.maximum(out1, 0.0)
    h2 = jnp.dot(r, w2_ref[...], preferred_element_type=jnp.float32)
    h2s_ref[...] = h2 * dinv


def _tc2(acc1, hs, degT, b1_2d, W2):
    return pl.pallas_call(
        _tc2_body,
        grid=(_GRID,),
        in_specs=[
            pl.BlockSpec((2, _BLK, 128), lambda i: (0, i, 0)),
            pl.BlockSpec((2, _BLK, 128), lambda i: (0, i, 0)),
            pl.BlockSpec((_BLK, 2), lambda i: (i, 0)),
            pl.BlockSpec((1, 256), lambda i: (0, 0)),
            pl.BlockSpec((256, 128), lambda i: (0, 0)),
        ],
        out_specs=pl.BlockSpec((_BLK, 128), lambda i: (i, 0)),
        out_shape=jax.ShapeDtypeStruct((NT, 128), jnp.float32),
    )(acc1, hs, degT, b1_2d, W2)


def _tc3_body(acc_ref, h2s_ref, deg_ref, b2_ref, out_ref):
    d = deg_ref[:, 0:1] + deg_ref[:, 1:2] + 1.0
    dinv = lax.rsqrt(d)
    u = acc_ref[0] + acc_ref[1] + h2s_ref[...]
    out_ref[...] = dinv * u + b2_ref[...]


def _tc3(acc2, h2s, degT, b2_2d):
    return pl.pallas_call(
        _tc3_body,
        grid=(_GRID,),
        in_specs=[
            pl.BlockSpec((2, _BLK, 128), lambda i: (0, i, 0)),
            pl.BlockSpec((_BLK, 128), lambda i: (i, 0)),
            pl.BlockSpec((_BLK, 2), lambda i: (i, 0)),
            pl.BlockSpec((1, 128), lambda i: (0, 0)),
        ],
        out_specs=pl.BlockSpec((_BLK, 128), lambda i: (i, 0)),
        out_shape=jax.ShapeDtypeStruct((NT, 128), jnp.float32),
    )(acc2, h2s, degT, b2_2d)


def kernel(x, edge_index, W1, b1, W2, b2):
    src = edge_index[0].astype(jnp.int32)
    dst = edge_index[1].astype(jnp.int32)

    pad = jnp.full((EP - E,), N, dtype=jnp.int32)
    src_p = jnp.concatenate([src, pad])
    dst_p = jnp.concatenate([dst, pad])
    src2 = jnp.stack([src_p, src_p + NT]).reshape(2, EROWS, 128)
    src_r = src_p.reshape(EROWS, 128)
    dst_r = dst_p.reshape(EROWS, 128)
    x_pad = jnp.pad(x, ((0, NT - N), (0, 0)))

    ones128 = jnp.ones((128,), jnp.float32)
    zeros1 = jnp.zeros((NODES_PER_TILE,), jnp.float32)
    zeros128 = jnp.zeros((NODES_PER_TILE, 128), jnp.float32)

    deg2 = _deg_kernel(dst_r, ones128, zeros1)
    degT = deg2.T

    hs = _tc1(x_pad, W1, degT)
    acc1 = _edge_kernel_l1(hs.reshape(2 * NT, 128), src2, dst_r, zeros128)
    h2s = _tc2(acc1, hs, degT, b1.reshape(1, 256), W2)
    acc2 = _edge_kernel_l2(h2s, src_r, dst_r, zeros128)
    out = _tc3(acc2, h2s, degT, b2.reshape(1, 128))
    return out[:N]


# speedup vs baseline: 8.9515x; 1.1000x over previous
import functools

import jax
import jax.numpy as jnp
from jax import lax
from jax.experimental import pallas as pl
from jax.experimental.pallas import tpu as pltpu
from jax.experimental.pallas import tpu_sc as plsc

N = 10000
NT = 10240
E = 320000
EP = 327680
EROWS = EP // 128
ROWS_PER_TILE = EROWS // 16
DEG_ROWS_PER_TILE = EROWS // 32
NODES_PER_TILE = NT // 16

_MESH = plsc.VectorSubcoreMesh(core_axis_name="c", subcore_axis_name="s")


@functools.partial(
    pl.kernel,
    out_type=jax.ShapeDtypeStruct((2, NT), jnp.float32),
    mesh=_MESH,
    scratch_types=[
        pltpu.VMEM_SHARED((NT,), jnp.float32),
        pltpu.VMEM((DEG_ROWS_PER_TILE, 128), jnp.int32),
        pltpu.VMEM((128,), jnp.float32),
    ],
)
def _deg_kernel(dst_hbm, ones_hbm, zeros1_hbm, deg_out, deg_sh, idx_v, ones_v):
    c = lax.axis_index("c")
    s = lax.axis_index("s")
    pltpu.sync_copy(zeros1_hbm, deg_sh.at[pl.ds(s * NODES_PER_TILE, NODES_PER_TILE)])
    row0 = c * (EROWS // 2) + s * DEG_ROWS_PER_TILE
    pltpu.sync_copy(dst_hbm.at[pl.ds(row0, DEG_ROWS_PER_TILE)], idx_v)
    pltpu.sync_copy(ones_hbm, ones_v)
    plsc.subcore_barrier()

    def body(j, carry):
        pltpu.sync_copy(ones_v, deg_sh.at[idx_v.at[j]], add=True)
        return carry

    lax.fori_loop(0, DEG_ROWS_PER_TILE, body, 0)
    plsc.subcore_barrier()
    pltpu.sync_copy(
        deg_sh.at[pl.ds(s * NODES_PER_TILE, NODES_PER_TILE)],
        deg_out.at[c, pl.ds(s * NODES_PER_TILE, NODES_PER_TILE)],
    )


SLAB = 16


def _make_edge_kernel(table_rows, edge_split):
    rows_per_tile = ROWS_PER_TILE if not edge_split else EROWS // 32
    nslab = rows_per_tile // SLAB

    @functools.partial(
        pl.kernel,
        out_type=jax.ShapeDtypeStruct((2, NT, 128), jnp.float32),
        mesh=_MESH,
        scratch_types=[
            pltpu.VMEM_SHARED((NT, 128), jnp.float32),
            pltpu.VMEM((SLAB, 128), jnp.int32),
            pltpu.VMEM((SLAB, 128), jnp.int32),
            pltpu.VMEM((2, 128, 128), jnp.float32),
            pltpu.SemaphoreType.DMA,
            pltpu.SemaphoreType.DMA,
        ],
    )
    def edge_kernel(table_hbm, src2_hbm, dst_hbm, zeros_hbm, acc_out,
                    acc_sh, src_idx, dst_idx, rows_v, sem0, sem1):
        c = lax.axis_index("c")
        s = lax.axis_index("s")
        pltpu.sync_copy(zeros_hbm, acc_sh.at[pl.ds(s * NODES_PER_TILE, NODES_PER_TILE)])
        plsc.subcore_barrier()
        if edge_split:
            r0 = c * (EROWS // 2) + s * rows_per_tile
        else:
            r0 = s * rows_per_tile

        def slab_body(t, carry):
            if edge_split:
                pltpu.sync_copy(src2_hbm.at[pl.ds(r0 + t * SLAB, SLAB)], src_idx)
            else:
                pltpu.sync_copy(src2_hbm.at[c, pl.ds(r0 + t * SLAB, SLAB)], src_idx)
            pltpu.sync_copy(dst_hbm.at[pl.ds(r0 + t * SLAB, SLAB)], dst_idx)
            pltpu.async_copy(table_hbm.at[src_idx.at[0]], rows_v.at[0], sem0)

            def body(jp, carry2):
                j = jp * 2
                pltpu.make_async_copy(table_hbm.at[src_idx.at[j]],
                                      rows_v.at[0], sem0).wait()
                pltpu.async_copy(table_hbm.at[src_idx.at[j + 1]],
                                 rows_v.at[1], sem1)
                pltpu.sync_copy(rows_v.at[0], acc_sh.at[dst_idx.at[j]], add=True)
                pltpu.make_async_copy(table_hbm.at[src_idx.at[j + 1]],
                                      rows_v.at[1], sem1).wait()

                @pl.when(j + 2 < SLAB)
                def _():
                    pltpu.async_copy(table_hbm.at[src_idx.at[j + 2]],
                                     rows_v.at[0], sem0)

                pltpu.sync_copy(rows_v.at[1], acc_sh.at[dst_idx.at[j + 1]], add=True)
                return carry2

            lax.fori_loop(0, SLAB // 2, body, 0)
            return carry

        lax.fori_loop(0, nslab, slab_body, 0)
        plsc.subcore_barrier()
        pltpu.sync_copy(
            acc_sh.at[pl.ds(s * NODES_PER_TILE, NODES_PER_TILE)],
            acc_out.at[c, pl.ds(s * NODES_PER_TILE, NODES_PER_TILE)],
        )

    return edge_kernel


_edge_kernel_l1 = _make_edge_kernel(2 * NT, edge_split=False)
_edge_kernel_l2 = _make_edge_kernel(NT, edge_split=True)


_BLK = 512
_GRID = NT // _BLK


def _tc1_body(x_ref, w1_ref, deg_ref, hs_ref):
    d = deg_ref[:, 0:1] + deg_ref[:, 1:2] + 1.0
    dinv = lax.rsqrt(d)
    h = jnp.dot(x_ref[...], w1_ref[...], preferred_element_type=jnp.float32)
    hsc = h * dinv
    hs_ref[0] = hsc[:, :128]
    hs_ref[1] = hsc[:, 128:]


def _tc1(x_pad, W1, degT):
    return pl.pallas_call(
        _tc1_body,
        grid=(_GRID,),
        in_specs=[
            pl.BlockSpec((_BLK, 128), lambda i: (i, 0)),
            pl.BlockSpec((128, 256), lambda i: (0, 0)),
            pl.BlockSpec((_BLK, 2), lambda i: (i, 0)),
        ],
        out_specs=pl.BlockSpec((2, _BLK, 128), lambda i: (0, i, 0)),
        out_shape=jax.ShapeDtypeStruct((2, NT, 128), jnp.float32),
    )(x_pad, W1, degT)


def _tc2_body(acc_ref, hs_ref, deg_ref, b1_ref, w2_ref, h2s_ref):
    d = deg_ref[:, 0:1] + deg_ref[:, 1:2] + 1.0
    dinv = lax.rsqrt(d)
    u = jnp.concatenate([acc_ref[0] + hs_ref[0], acc_ref[1] + hs_ref[1]], axis=1)
    out1 = dinv * u + b1_ref[...]
    r = jnp.maximum(out1, 0.0)
    h2 = jnp.dot(r, w2_ref[...], preferred_element_type=jnp.float32)
    h2s_ref[...] = h2 * dinv


def _tc2(acc1, hs, degT, b1_2d, W2):
    return pl.pallas_call(
        _tc2_body,
        grid=(_GRID,),
        in_specs=[
            pl.BlockSpec((2, _BLK, 128), lambda i: (0, i, 0)),
            pl.BlockSpec((2, _BLK, 128), lambda i: (0, i, 0)),
            pl.BlockSpec((_BLK, 2), lambda i: (i, 0)),
            pl.BlockSpec((1, 256), lambda i: (0, 0)),
            pl.BlockSpec((256, 128), lambda i: (0, 0)),
        ],
        out_specs=pl.BlockSpec((_BLK, 128), lambda i: (i, 0)),
        out_shape=jax.ShapeDtypeStruct((NT, 128), jnp.float32),
    )(acc1, hs, degT, b1_2d, W2)


def _tc3_body(acc_ref, h2s_ref, deg_ref, b2_ref, out_ref):
    d = deg_ref[:, 0:1] + deg_ref[:, 1:2] + 1.0
    dinv = lax.rsqrt(d)
    u = acc_ref[0] + acc_ref[1] + h2s_ref[...]
    out_ref[...] = dinv * u + b2_ref[...]


def _tc3(acc2, h2s, degT, b2_2d):
    return pl.pallas_call(
        _tc3_body,
        grid=(_GRID,),
        in_specs=[
            pl.BlockSpec((2, _BLK, 128), lambda i: (0, i, 0)),
            pl.BlockSpec((_BLK, 128), lambda i: (i, 0)),
            pl.BlockSpec((_BLK, 2), lambda i: (i, 0)),
            pl.BlockSpec((1, 128), lambda i: (0, 0)),
        ],
        out_specs=pl.BlockSpec((_BLK, 128), lambda i: (i, 0)),
        out_shape=jax.ShapeDtypeStruct((NT, 128), jnp.float32),
    )(acc2, h2s, degT, b2_2d)


def kernel(x, edge_index, W1, b1, W2, b2):
    src = edge_index[0].astype(jnp.int32)
    dst = edge_index[1].astype(jnp.int32)

    pad = jnp.full((EP - E,), N, dtype=jnp.int32)
    src_p = jnp.concatenate([src, pad])
    dst_p = jnp.concatenate([dst, pad])
    src2 = jnp.stack([src_p, src_p + NT]).reshape(2, EROWS, 128)
    src_r = src_p.reshape(EROWS, 128)
    dst_r = dst_p.reshape(EROWS, 128)
    x_pad = jnp.pad(x, ((0, NT - N), (0, 0)))

    ones128 = jnp.ones((128,), jnp.float32)
    zeros1 = jnp.zeros((NODES_PER_TILE,), jnp.float32)
    zeros128 = jnp.zeros((NODES_PER_TILE, 128), jnp.float32)

    deg2 = _deg_kernel(dst_r, ones128, zeros1)
    degT = deg2.T

    hs = _tc1(x_pad, W1, degT)
    acc1 = _edge_kernel_l1(hs.reshape(2 * NT, 128), src2, dst_r, zeros128)
    h2s = _tc2(acc1, hs, degT, b1.reshape(1, 256), W2)
    acc2 = _edge_kernel_l2(h2s, src_r, dst_r, zeros128)
    out = _tc3(acc2, h2s, degT, b2.reshape(1, 128))
    return out[:N]


# speedup vs baseline: 9.8365x; 1.0989x over previous
import functools

import jax
import jax.numpy as jnp
from jax import lax
from jax.experimental import pallas as pl
from jax.experimental.pallas import tpu as pltpu
from jax.experimental.pallas import tpu_sc as plsc

N = 10000
NT = 10240
E = 320000
EP = 327680
EROWS = EP // 128
ROWS_PER_TILE = EROWS // 16
DEG_ROWS_PER_TILE = EROWS // 32
NODES_PER_TILE = NT // 16

_MESH = plsc.VectorSubcoreMesh(core_axis_name="c", subcore_axis_name="s")


@functools.partial(
    pl.kernel,
    out_type=jax.ShapeDtypeStruct((2, NT), jnp.float32),
    mesh=_MESH,
    scratch_types=[
        pltpu.VMEM_SHARED((NT,), jnp.float32),
        pltpu.VMEM((DEG_ROWS_PER_TILE, 128), jnp.int32),
        pltpu.VMEM((128,), jnp.float32),
    ],
)
def _deg_kernel(dst_hbm, ones_hbm, zeros1_hbm, deg_out, deg_sh, idx_v, ones_v):
    c = lax.axis_index("c")
    s = lax.axis_index("s")
    pltpu.sync_copy(zeros1_hbm, deg_sh.at[pl.ds(s * NODES_PER_TILE, NODES_PER_TILE)])
    row0 = c * (EROWS // 2) + s * DEG_ROWS_PER_TILE
    pltpu.sync_copy(dst_hbm.at[pl.ds(row0, DEG_ROWS_PER_TILE)], idx_v)
    pltpu.sync_copy(ones_hbm, ones_v)
    plsc.subcore_barrier()

    def body(j, carry):
        pltpu.sync_copy(ones_v, deg_sh.at[idx_v.at[j]], add=True)
        return carry

    lax.fori_loop(0, DEG_ROWS_PER_TILE, body, 0)
    plsc.subcore_barrier()
    pltpu.sync_copy(
        deg_sh.at[pl.ds(s * NODES_PER_TILE, NODES_PER_TILE)],
        deg_out.at[c, pl.ds(s * NODES_PER_TILE, NODES_PER_TILE)],
    )


SLAB = 32


def _make_edge_kernel(table_rows, edge_split):
    rows_per_tile = ROWS_PER_TILE if not edge_split else EROWS // 32
    nslab = rows_per_tile // SLAB

    @functools.partial(
        pl.kernel,
        out_type=jax.ShapeDtypeStruct((2, NT, 128), jnp.float32),
        mesh=_MESH,
        scratch_types=[
            pltpu.VMEM_SHARED((NT, 128), jnp.float32),
            pltpu.VMEM((SLAB, 128), jnp.int32),
            pltpu.VMEM((SLAB, 128), jnp.int32),
            pltpu.VMEM((2, 128, 128), jnp.float32),
            pltpu.SemaphoreType.DMA,
            pltpu.SemaphoreType.DMA,
            pltpu.SemaphoreType.DMA,
            pltpu.SemaphoreType.DMA,
        ],
    )
    def edge_kernel(table_hbm, src2_hbm, dst_hbm, zeros_hbm, acc_out,
                    acc_sh, src_idx, dst_idx, rows_v, semg0, semg1, sems0, sems1):
        c = lax.axis_index("c")
        s = lax.axis_index("s")
        pltpu.sync_copy(zeros_hbm, acc_sh.at[pl.ds(s * NODES_PER_TILE, NODES_PER_TILE)])
        plsc.subcore_barrier()
        if edge_split:
            r0 = c * (EROWS // 2) + s * rows_per_tile
        else:
            r0 = s * rows_per_tile

        def slab_body(t, carry):
            if edge_split:
                pltpu.sync_copy(src2_hbm.at[pl.ds(r0 + t * SLAB, SLAB)], src_idx)
            else:
                pltpu.sync_copy(src2_hbm.at[c, pl.ds(r0 + t * SLAB, SLAB)], src_idx)
            pltpu.sync_copy(dst_hbm.at[pl.ds(r0 + t * SLAB, SLAB)], dst_idx)
            pltpu.async_copy(table_hbm.at[src_idx.at[0]], rows_v.at[0], semg0)
            pltpu.async_copy(table_hbm.at[src_idx.at[1]], rows_v.at[1], semg1)

            def body(jp, carry2):
                j = jp * 2
                pltpu.make_async_copy(table_hbm.at[src_idx.at[j]],
                                      rows_v.at[0], semg0).wait()
                pltpu.async_copy(rows_v.at[0], acc_sh.at[dst_idx.at[j]],
                                 sems0, add=True)
                pltpu.make_async_copy(table_hbm.at[src_idx.at[j + 1]],
                                      rows_v.at[1], semg1).wait()
                pltpu.async_copy(rows_v.at[1], acc_sh.at[dst_idx.at[j + 1]],
                                 sems1, add=True)
                pltpu.make_async_copy(rows_v.at[0], acc_sh.at[dst_idx.at[j]],
                                      sems0).wait()

                @pl.when(j + 2 < SLAB)
                def _():
                    pltpu.async_copy(table_hbm.at[src_idx.at[j + 2]],
                                     rows_v.at[0], semg0)

                pltpu.make_async_copy(rows_v.at[1], acc_sh.at[dst_idx.at[j + 1]],
                                      sems1).wait()

                @pl.when(j + 3 < SLAB)
                def _():
                    pltpu.async_copy(table_hbm.at[src_idx.at[j + 3]],
                                     rows_v.at[1], semg1)

                return carry2

            lax.fori_loop(0, SLAB // 2, body, 0)
            return carry

        lax.fori_loop(0, nslab, slab_body, 0)
        plsc.subcore_barrier()
        pltpu.sync_copy(
            acc_sh.at[pl.ds(s * NODES_PER_TILE, NODES_PER_TILE)],
            acc_out.at[c, pl.ds(s * NODES_PER_TILE, NODES_PER_TILE)],
        )

    return edge_kernel


_edge_kernel_l1 = _make_edge_kernel(2 * NT, edge_split=False)
_edge_kernel_l2 = _make_edge_kernel(NT, edge_split=True)


_BLK = 512
_GRID = NT // _BLK


def _tc1_body(x_ref, w1_ref, deg_ref, hs_ref):
    d = deg_ref[:, 0:1] + deg_ref[:, 1:2] + 1.0
    dinv = lax.rsqrt(d)
    h = jnp.dot(x_ref[...], w1_ref[...], preferred_element_type=jnp.float32)
    hsc = h * dinv
    hs_ref[0] = hsc[:, :128]
    hs_ref[1] = hsc[:, 128:]


def _tc1(x_pad, W1, degT):
    return pl.pallas_call(
        _tc1_body,
        grid=(_GRID,),
        in_specs=[
            pl.BlockSpec((_BLK, 128), lambda i: (i, 0)),
            pl.BlockSpec((128, 256), lambda i: (0, 0)),
            pl.BlockSpec((_BLK, 2), lambda i: (i, 0)),
        ],
        out_specs=pl.BlockSpec((2, _BLK, 128), lambda i: (0, i, 0)),
        out_shape=jax.ShapeDtypeStruct((2, NT, 128), jnp.float32),
    )(x_pad, W1, degT)


def _tc2_body(acc_ref, hs_ref, deg_ref, b1_ref, w2_ref, h2s_ref):
    d = deg_ref[:, 0:1] + deg_ref[:, 1:2] + 1.0
    dinv = lax.rsqrt(d)
    u = jnp.concatenate([acc_ref[0] + hs_ref[0], acc_ref[1] + hs_ref[1]], axis=1)
    out1 = dinv * u + b1_ref[...]
    r = jnp.maximum(out1, 0.0)
    h2 = jnp.dot(r, w2_ref[...], preferred_element_type=jnp.float32)
    h2s_ref[...] = h2 * dinv


def _tc2(acc1, hs, degT, b1_2d, W2):
    return pl.pallas_call(
        _tc2_body,
        grid=(_GRID,),
        in_specs=[
            pl.BlockSpec((2, _BLK, 128), lambda i: (0, i, 0)),
            pl.BlockSpec((2, _BLK, 128), lambda i: (0, i, 0)),
            pl.BlockSpec((_BLK, 2), lambda i: (i, 0)),
            pl.BlockSpec((1, 256), lambda i: (0, 0)),
            pl.BlockSpec((256, 128), lambda i: (0, 0)),
        ],
        out_specs=pl.BlockSpec((_BLK, 128), lambda i: (i, 0)),
        out_shape=jax.ShapeDtypeStruct((NT, 128), jnp.float32),
    )(acc1, hs, degT, b1_2d, W2)


def _tc3_body(acc_ref, h2s_ref, deg_ref, b2_ref, out_ref):
    d = deg_ref[:, 0:1] + deg_ref[:, 1:2] + 1.0
    dinv = lax.rsqrt(d)
    u = acc_ref[0] + acc_ref[1] + h2s_ref[...]
    out_ref[...] = dinv * u + b2_ref[...]


def _tc3(acc2, h2s, degT, b2_2d):
    return pl.pallas_call(
        _tc3_body,
        grid=(_GRID,),
        in_specs=[
            pl.BlockSpec((2, _BLK, 128), lambda i: (0, i, 0)),
            pl.BlockSpec((_BLK, 128), lambda i: (i, 0)),
            pl.BlockSpec((_BLK, 2), lambda i: (i, 0)),
            pl.BlockSpec((1, 128), lambda i: (0, 0)),
        ],
        out_specs=pl.BlockSpec((_BLK, 128), lambda i: (i, 0)),
        out_shape=jax.ShapeDtypeStruct((NT, 128), jnp.float32),
    )(acc2, h2s, degT, b2_2d)


def kernel(x, edge_index, W1, b1, W2, b2):
    src = edge_index[0].astype(jnp.int32)
    dst = edge_index[1].astype(jnp.int32)

    pad = jnp.full((EP - E,), N, dtype=jnp.int32)
    src_p = jnp.concatenate([src, pad])
    dst_p = jnp.concatenate([dst, pad])
    src2 = jnp.stack([src_p, src_p + NT]).reshape(2, EROWS, 128)
    src_r = src_p.reshape(EROWS, 128)
    dst_r = dst_p.reshape(EROWS, 128)
    x_pad = jnp.pad(x, ((0, NT - N), (0, 0)))

    ones128 = jnp.ones((128,), jnp.float32)
    zeros1 = jnp.zeros((NODES_PER_TILE,), jnp.float32)
    zeros128 = jnp.zeros((NODES_PER_TILE, 128), jnp.float32)

    deg2 = _deg_kernel(dst_r, ones128, zeros1)
    degT = deg2.T

    hs = _tc1(x_pad, W1, degT)
    acc1 = _edge_kernel_l1(hs.reshape(2 * NT, 128), src2, dst_r, zeros128)
    h2s = _tc2(acc1, hs, degT, b1.reshape(1, 256), W2)
    acc2 = _edge_kernel_l2(h2s, src_r, dst_r, zeros128)
    out = _tc3(acc2, h2s, degT, b2.reshape(1, 128))
    return out[:N]
